# baseline (speedup 1.0000x reference)
import functools

import jax
import jax.numpy as jnp
from jax import lax
from jax.experimental import pallas as pl
from jax.experimental.pallas import tpu as pltpu
from jax.experimental.pallas import tpu_sc as plsc

_N = 10000
_E = 320000
_D_IN = 128
_D_HID = 16
_NC = 2
_NS = 16
_NW = _NC * _NS
_EPT = _E // _NW
_EPH = _E // 2
_L1_CHUNKS = _EPH // _EPT
_UNROLL = 8

_MESH = dict(core_axis_name="c", subcore_axis_name="s", num_cores=_NC,
             num_subcores=_NS)


def _wid():
    return lax.axis_index("s") * _NC + lax.axis_index("c")


def _zero(acc_v, n):
    @plsc.parallel_loop(0, n // 16, 1, unroll=_UNROLL)
    def zb(i):
        acc_v[pl.ds(i * 16, 16)] = jnp.zeros((16,), jnp.float32)


def _unpack_rc(rcv):
    rowi = lax.bitwise_and(rcv, jnp.int32(0xFFFF))
    coli = lax.shift_right_logical(rcv, jnp.int32(16))
    return rowi, coli


def _sc_deg_body(col_hbm, ew_hbm, out_hbm, col_v, ew_v, acc_v, sem):
    wid = _wid()
    base = wid * _EPT
    h1 = pltpu.async_copy(col_hbm.at[pl.ds(base, _EPT)], col_v, sem)
    h2 = pltpu.async_copy(ew_hbm.at[pl.ds(base, _EPT)], ew_v, sem)
    _zero(acc_v, _N)
    h1.wait()
    h2.wait()

    @plsc.parallel_loop(0, _EPT // 16, 1, unroll=_UNROLL)
    def body(i):
        s = pl.ds(i * 16, 16)
        plsc.addupdate_scatter(acc_v, [col_v[s]], ew_v[s])
    pltpu.sync_copy(acc_v, out_hbm.at[wid])


_sc_deg = pl.kernel(
    _sc_deg_body,
    out_type=jax.ShapeDtypeStruct((_NW, _N), jnp.float32),
    mesh=plsc.VectorSubcoreMesh(**_MESH),
    compiler_params=pltpu.CompilerParams(needs_layout_passes=False),
    scratch_types=[
        pltpu.VMEM((_EPT,), jnp.int32),
        pltpu.VMEM((_EPT,), jnp.float32),
        pltpu.VMEM((_N,), jnp.float32),
        pltpu.SemaphoreType.DMA,
    ],
)


def _sc_l1_body(rc_hbm, ew_hbm, g1t_hbm, out_hbm,
                rc_v0, rc_v1, rc_v2, ew_v0, ew_v1, ew_v2, tab_v, acc_v,
                sem0, sem1, sem2):
    wid = _wid()
    d = wid % _D_HID
    half = wid // _D_HID
    bufs = ((rc_v0, ew_v0, sem0), (rc_v1, ew_v1, sem1), (rc_v2, ew_v2, sem2))

    def start(c, b):
        base = half * _EPH + c * _EPT
        rc_v, ew_v, sem = bufs[b]
        h1 = pltpu.async_copy(rc_hbm.at[pl.ds(base, _EPT)], rc_v, sem)
        h2 = pltpu.async_copy(ew_hbm.at[pl.ds(base, _EPT)], ew_v, sem)
        return (h1, h2)

    pend = start(0, 0)
    pltpu.sync_copy(g1t_hbm.at[d], tab_v)
    _zero(acc_v, _N)
    for c in range(_L1_CHUNKS):
        b = c % 3
        rc_v, ew_v, _ = bufs[b]
        for h in pend:
            h.wait()
        if c + 1 < _L1_CHUNKS:
            pend = start(c + 1, (c + 1) % 3)

        @plsc.parallel_loop(0, _EPT // 16, 1, unroll=_UNROLL)
        def body(i):
            s = pl.ds(i * 16, 16)
            rowi, coli = _unpack_rc(rc_v[s])
            vals = plsc.load_gather(tab_v, [rowi]) * ew_v[s]
            plsc.addupdate_scatter(acc_v, [coli], vals)
    pltpu.sync_copy(acc_v, out_hbm.at[wid])


_sc_l1 = pl.kernel(
    _sc_l1_body,
    out_type=jax.ShapeDtypeStruct((_NW, _N), jnp.float32),
    mesh=plsc.VectorSubcoreMesh(**_MESH),
    compiler_params=pltpu.CompilerParams(needs_layout_passes=False),
    scratch_types=[
        pltpu.VMEM((_EPT,), jnp.int32),
        pltpu.VMEM((_EPT,), jnp.int32),
        pltpu.VMEM((_EPT,), jnp.int32),
        pltpu.VMEM((_EPT,), jnp.float32),
        pltpu.VMEM((_EPT,), jnp.float32),
        pltpu.VMEM((_EPT,), jnp.float32),
        pltpu.VMEM((_N,), jnp.float32),
        pltpu.VMEM((_N,), jnp.float32),
        pltpu.SemaphoreType.DMA,
        pltpu.SemaphoreType.DMA,
        pltpu.SemaphoreType.DMA,
    ],
)


def _sc_l2f_body(rc_hbm, ew_hbm, g2b_hbm, dis_hbm, out_hbm,
                 rc_a, rc_b, ew_a, ew_b, tab_v, acc_v, red_v, g2l_v, disl_v,
                 out_v, shared, sem_a, sem_b, sem_s):
    c = lax.axis_index("c")
    s = lax.axis_index("s")
    base = s * (2 * _EPT)
    ha1 = pltpu.async_copy(rc_hbm.at[pl.ds(base, _EPT)], rc_a, sem_a)
    ha2 = pltpu.async_copy(ew_hbm.at[pl.ds(base, _EPT)], ew_a, sem_a)
    hb1 = pltpu.async_copy(rc_hbm.at[pl.ds(base + _EPT, _EPT)], rc_b, sem_b)
    hb2 = pltpu.async_copy(ew_hbm.at[pl.ds(base + _EPT, _EPT)], ew_b, sem_b)
    h3 = pltpu.async_copy(g2b_hbm, tab_v, sem_s)
    _zero(acc_v, _N)
    ha1.wait()
    ha2.wait()
    h3.wait()
    for rc_v, ew_v in ((rc_a, ew_a), (rc_b, ew_b)):
        if rc_v is rc_b:
            hb1.wait()
            hb2.wait()

        @plsc.parallel_loop(0, _EPT // 16, 1, unroll=_UNROLL)
        def body(i):
            sl = pl.ds(i * 16, 16)
            rowi, coli = _unpack_rc(rc_v[sl])
            vals = plsc.load_gather(tab_v, [rowi]) * ew_v[sl]
            plsc.addupdate_scatter(acc_v, [coli], vals)

    stg = []
    for k in range(16):
        ln = 640 if k < 15 else 400
        stg.append(pltpu.async_copy(acc_v.at[pl.ds(k * 640, ln)],
                                    shared.at[pl.ds((k * 16 + s) * 640, ln)],
                                    sem_s))
    for h in stg:
        h.wait()
    plsc.subcore_barrier()

    mine = jnp.logical_or(jnp.logical_and(c == 0, s < 8),
                          jnp.logical_and(c == 1, s >= 8))
    node0 = s * 640

    def emit(nch, ln):
        pltpu.sync_copy(shared.at[pl.ds(s * 10240, 10240)], red_v)
        pltpu.sync_copy(g2b_hbm.at[pl.ds(node0, ln)], g2l_v.at[pl.ds(0, ln)])
        pltpu.sync_copy(dis_hbm.at[pl.ds(node0, ln)], disl_v.at[pl.ds(0, ln)])

        @plsc.parallel_loop(0, nch, 1, unroll=4)
        def red(j):
            sl = pl.ds(j * 16, 16)
            tot = red_v[pl.ds(j * 16, 16)]
            for p in range(1, 16):
                tot = tot + red_v[pl.ds(p * 640 + j * 16, 16)]
            out_v[sl] = disl_v[sl] * (tot + g2l_v[sl])
        pltpu.sync_copy(out_v.at[pl.ds(0, ln)], out_hbm.at[pl.ds(node0, ln)])

    @pl.when(jnp.logical_and(mine, s != 15))
    def _():
        emit(40, 640)

    @pl.when(jnp.logical_and(mine, s == 15))
    def _():
        emit(25, 400)


_sc_l2f = pl.kernel(
    _sc_l2f_body,
    out_type=jax.ShapeDtypeStruct((_N,), jnp.float32),
    mesh=plsc.VectorSubcoreMesh(**_MESH),
    compiler_params=pltpu.CompilerParams(needs_layout_passes=False),
    scratch_types=[
        pltpu.VMEM((_EPT,), jnp.int32),
        pltpu.VMEM((_EPT,), jnp.int32),
        pltpu.VMEM((_EPT,), jnp.float32),
        pltpu.VMEM((_EPT,), jnp.float32),
        pltpu.VMEM((_N,), jnp.float32),
        pltpu.VMEM((_N,), jnp.float32),
        pltpu.VMEM((10240,), jnp.float32),
        pltpu.VMEM((640,), jnp.float32),
        pltpu.VMEM((640,), jnp.float32),
        pltpu.VMEM((640,), jnp.float32),
        pltpu.VMEM_SHARED((163840,), jnp.float32),
        pltpu.SemaphoreType.DMA,
        pltpu.SemaphoreType.DMA,
        pltpu.SemaphoreType.DMA,
    ],
)


def _tc_prep_body(x_ref, w1_ref, row_ref, col_ref, degp_ref,
                  g1t_ref, dis_ref, rc_ref):
    h = jnp.dot(x_ref[...], w1_ref[...], preferred_element_type=jnp.float32)
    deg = 1.0 + jnp.sum(degp_ref[...], axis=0, keepdims=True)
    dis = lax.rsqrt(deg)
    g1t_ref[...] = h.T * dis
    dis_ref[...] = dis
    rc_ref[...] = row_ref[...] + col_ref[...] * jnp.int32(65536)


_tc_prep = pl.pallas_call(
    _tc_prep_body,
    out_shape=[
        jax.ShapeDtypeStruct((_D_HID, _N), jnp.float32),
        jax.ShapeDtypeStruct((1, _N), jnp.float32),
        jax.ShapeDtypeStruct((_E // 128, 128), jnp.int32),
    ],
)


def _tc_mid_body(accp_ref, g1t_ref, dis_ref, b1_ref, w2t_ref, b2_ref,
                 g2b_ref):
    accp = accp_ref[...]
    acc = accp[0:_D_HID] + accp[_D_HID:_NW]
    dis = dis_ref[...]
    pre = dis * (acc + g1t_ref[...]) + b1_ref[...]
    r = jnp.maximum(pre, 0.0)
    h2 = jnp.dot(w2t_ref[...], r, preferred_element_type=jnp.float32)
    g2b_ref[...] = dis * h2 + b2_ref[...] / dis


_tc_mid = pl.pallas_call(
    _tc_mid_body,
    out_shape=jax.ShapeDtypeStruct((1, _N), jnp.float32),
)


def kernel(x, edge_index, edge_weight, W1, b1, W2, b2):
    ei = edge_index.astype(jnp.int32)
    row = ei[0]
    col = ei[1]
    ew = edge_weight.astype(jnp.float32)

    degp = _sc_deg(col, ew)
    g1t, dis, rc2d = _tc_prep(x, W1, row.reshape(_E // 128, 128),
                              col.reshape(_E // 128, 128), degp)
    rc = rc2d.reshape(_E)
    accp1 = _sc_l1(rc, ew, g1t)
    g2b = _tc_mid(accp1, g1t, dis, b1.reshape(_D_HID, 1),
                  W2.reshape(1, _D_HID), b2.reshape(1, 1))
    out = _sc_l2f(rc, ew, g2b.reshape(_N), dis.reshape(_N))
    return out.reshape(_N, 1)


# speedup vs baseline: 1.0588x; 1.0588x over previous
import functools

import jax
import jax.numpy as jnp
from jax import lax
from jax.experimental import pallas as pl
from jax.experimental.pallas import tpu as pltpu
from jax.experimental.pallas import tpu_sc as plsc

_N = 10000
_E = 320000
_D_IN = 128
_D_HID = 16
_NC = 2
_NS = 16
_NW = _NC * _NS
_EPT = _E // _NW
_EPH = _E // 2
_L1_CHUNKS = _EPH // _EPT
_UNROLL = 8

_MESH = dict(core_axis_name="c", subcore_axis_name="s", num_cores=_NC,
             num_subcores=_NS)


def _wid():
    return lax.axis_index("s") * _NC + lax.axis_index("c")


def _zero(acc_v, n):
    @plsc.parallel_loop(0, n // 16, 1, unroll=_UNROLL)
    def zb(i):
        acc_v[pl.ds(i * 16, 16)] = jnp.zeros((16,), jnp.float32)


def _unpack_rc(rcv):
    rowi = lax.bitwise_and(rcv, jnp.int32(0xFFFF))
    coli = lax.shift_right_logical(rcv, jnp.int32(16))
    return rowi, coli


def _sc_deg_body(col_hbm, ew_hbm, out_hbm, col_v, ew_v, acc_v, sem):
    wid = _wid()
    base = wid * _EPT
    h1 = pltpu.async_copy(col_hbm.at[pl.ds(base, _EPT)], col_v, sem)
    h2 = pltpu.async_copy(ew_hbm.at[pl.ds(base, _EPT)], ew_v, sem)
    _zero(acc_v, _N)
    h1.wait()
    h2.wait()

    @plsc.parallel_loop(0, _EPT // 16, 1, unroll=_UNROLL)
    def body(i):
        s = pl.ds(i * 16, 16)
        plsc.addupdate_scatter(acc_v, [col_v[s]], ew_v[s])
    pltpu.sync_copy(acc_v, out_hbm.at[wid])


_sc_deg = pl.kernel(
    _sc_deg_body,
    out_type=jax.ShapeDtypeStruct((_NW, _N), jnp.float32),
    mesh=plsc.VectorSubcoreMesh(**_MESH),
    compiler_params=pltpu.CompilerParams(needs_layout_passes=False),
    scratch_types=[
        pltpu.VMEM((_EPT,), jnp.int32),
        pltpu.VMEM((_EPT,), jnp.float32),
        pltpu.VMEM((_N,), jnp.float32),
        pltpu.SemaphoreType.DMA,
    ],
)


def _sc_l1_body(rc_hbm, ew_hbm, g1t_hbm, out_hbm,
                rc_v0, rc_v1, rc_v2, ew_v0, ew_v1, ew_v2, tab_v, acc_v,
                sem0, sem1, sem2):
    wid = _wid()
    d = wid % _D_HID
    half = wid // _D_HID
    bufs = ((rc_v0, ew_v0, sem0), (rc_v1, ew_v1, sem1), (rc_v2, ew_v2, sem2))

    def start(c, b):
        base = half * _EPH + c * _EPT
        rc_v, ew_v, sem = bufs[b]
        h1 = pltpu.async_copy(rc_hbm.at[pl.ds(base, _EPT)], rc_v, sem)
        h2 = pltpu.async_copy(ew_hbm.at[pl.ds(base, _EPT)], ew_v, sem)
        return (h1, h2)

    pend = start(0, 0)
    pltpu.sync_copy(g1t_hbm.at[d], tab_v)
    _zero(acc_v, _N)
    for c in range(_L1_CHUNKS):
        b = c % 3
        rc_v, ew_v, _ = bufs[b]
        for h in pend:
            h.wait()
        if c + 1 < _L1_CHUNKS:
            pend = start(c + 1, (c + 1) % 3)

        @plsc.parallel_loop(0, _EPT // 16, 1, unroll=_UNROLL)
        def body(i):
            s = pl.ds(i * 16, 16)
            rowi, coli = _unpack_rc(rc_v[s])
            vals = plsc.load_gather(tab_v, [rowi]) * ew_v[s]
            plsc.addupdate_scatter(acc_v, [coli], vals)
    pltpu.sync_copy(acc_v, out_hbm.at[wid])


_sc_l1 = pl.kernel(
    _sc_l1_body,
    out_type=jax.ShapeDtypeStruct((_NW, _N), jnp.float32),
    mesh=plsc.VectorSubcoreMesh(**_MESH),
    compiler_params=pltpu.CompilerParams(needs_layout_passes=False),
    scratch_types=[
        pltpu.VMEM((_EPT,), jnp.int32),
        pltpu.VMEM((_EPT,), jnp.int32),
        pltpu.VMEM((_EPT,), jnp.int32),
        pltpu.VMEM((_EPT,), jnp.float32),
        pltpu.VMEM((_EPT,), jnp.float32),
        pltpu.VMEM((_EPT,), jnp.float32),
        pltpu.VMEM((_N,), jnp.float32),
        pltpu.VMEM((_N,), jnp.float32),
        pltpu.SemaphoreType.DMA,
        pltpu.SemaphoreType.DMA,
        pltpu.SemaphoreType.DMA,
    ],
)


def _sc_l2_body(rc_hbm, ew_hbm, g2_hbm, out_hbm,
                rc_v, ew_v, tab_v, acc_v, sem):
    wid = _wid()
    base = wid * _EPT
    h1 = pltpu.async_copy(rc_hbm.at[pl.ds(base, _EPT)], rc_v, sem)
    h2 = pltpu.async_copy(ew_hbm.at[pl.ds(base, _EPT)], ew_v, sem)
    h3 = pltpu.async_copy(g2_hbm, tab_v, sem)
    _zero(acc_v, _N)
    h1.wait()
    h2.wait()
    h3.wait()

    @plsc.parallel_loop(0, _EPT // 16, 1, unroll=_UNROLL)
    def body(i):
        s = pl.ds(i * 16, 16)
        rowi, coli = _unpack_rc(rc_v[s])
        vals = plsc.load_gather(tab_v, [rowi]) * ew_v[s]
        plsc.addupdate_scatter(acc_v, [coli], vals)
    pltpu.sync_copy(acc_v, out_hbm.at[wid])


_sc_l2 = pl.kernel(
    _sc_l2_body,
    out_type=jax.ShapeDtypeStruct((_NW, _N), jnp.float32),
    mesh=plsc.VectorSubcoreMesh(**_MESH),
    compiler_params=pltpu.CompilerParams(needs_layout_passes=False),
    scratch_types=[
        pltpu.VMEM((_EPT,), jnp.int32),
        pltpu.VMEM((_EPT,), jnp.float32),
        pltpu.VMEM((_N,), jnp.float32),
        pltpu.VMEM((_N,), jnp.float32),
        pltpu.SemaphoreType.DMA,
    ],
)


def _tc_prep_body(x_ref, w1_ref, row_ref, col_ref, degp_ref,
                  g1t_ref, dis_ref, rc_ref):
    h = jnp.dot(x_ref[...], w1_ref[...], preferred_element_type=jnp.float32)
    deg = 1.0 + jnp.sum(degp_ref[...], axis=0, keepdims=True)
    dis = lax.rsqrt(deg)
    g1t_ref[...] = h.T * dis
    dis_ref[...] = dis
    rc_ref[...] = row_ref[...] + col_ref[...] * jnp.int32(65536)


_tc_prep = pl.pallas_call(
    _tc_prep_body,
    out_shape=[
        jax.ShapeDtypeStruct((_D_HID, _N), jnp.float32),
        jax.ShapeDtypeStruct((1, _N), jnp.float32),
        jax.ShapeDtypeStruct((_E // 128, 128), jnp.int32),
    ],
)


def _tc_mid_body(accp_ref, g1t_ref, dis_ref, b1_ref, w2t_ref, g2_ref):
    accp = accp_ref[...]
    acc = accp[0:_D_HID] + accp[_D_HID:_NW]
    dis = dis_ref[...]
    pre = dis * (acc + g1t_ref[...]) + b1_ref[...]
    r = jnp.maximum(pre, 0.0)
    h2 = jnp.dot(w2t_ref[...], r, preferred_element_type=jnp.float32)
    g2_ref[...] = dis * h2


_tc_mid = pl.pallas_call(
    _tc_mid_body,
    out_shape=jax.ShapeDtypeStruct((1, _N), jnp.float32),
)


def _tc_fin_body(accp_ref, g2_ref, dis_ref, b2_ref, out_ref):
    acc = jnp.sum(accp_ref[...], axis=0, keepdims=True) + g2_ref[...]
    out_ref[...] = dis_ref[...] * acc + b2_ref[...]


_tc_fin = pl.pallas_call(
    _tc_fin_body,
    out_shape=jax.ShapeDtypeStruct((1, _N), jnp.float32),
)


def kernel(x, edge_index, edge_weight, W1, b1, W2, b2):
    ei = edge_index.astype(jnp.int32)
    row = ei[0]
    col = ei[1]
    ew = edge_weight.astype(jnp.float32)

    degp = _sc_deg(col, ew)
    g1t, dis, rc2d = _tc_prep(x, W1, row.reshape(_E // 128, 128),
                              col.reshape(_E // 128, 128), degp)
    rc = rc2d.reshape(_E)
    accp1 = _sc_l1(rc, ew, g1t)
    g2 = _tc_mid(accp1, g1t, dis, b1.reshape(_D_HID, 1), W2.reshape(1, _D_HID))
    accp2 = _sc_l2(rc, ew, g2.reshape(_N))
    out = _tc_fin(accp2, g2, dis, b2.reshape(1, 1))
    return out.reshape(_N, 1)
